# Initial kernel scaffold; baseline (speedup 1.0000x reference)
import jax
import jax.numpy as jnp
from jax.experimental import pallas as pl


def kernel(x_nchw, conv10_w, conv10_b, conv11_w, conv11_b, conv20_w, conv20_b, conv21_w, conv21_b, conv30_w, conv30_b, conv31_w, conv31_b, conv32_w, conv32_b, conv40_w, conv40_b, conv41_w, conv41_b, conv42_w, conv42_b, fc1_w, fc1_b, fc2_w, fc2_b, fc3_w, fc3_b):
    raise NotImplementedError("write your pallas kernel here")


# speedup vs baseline: 1.1613x; 1.1613x over previous
import functools

import jax
import jax.numpy as jnp
from jax import lax
from jax.experimental import pallas as pl
from jax.experimental.pallas import tpu as pltpu

_VMEM = 56 << 20
_BF = jnp.bfloat16


def _flat(x4, Wp, Q, L):
    B, H, W, C = x4.shape
    xr = jnp.pad(x4, ((0, 0), (0, 0), (0, Wp - W), (0, 0)))
    xr = xr.reshape(B, H * Wp, C)
    return jnp.pad(xr, ((0, 0), (Q, L - Q - H * Wp), (0, 0)))


def _make_stage_body(H, W, Q, nconv):
    Wp = W + 2
    n = H * Wp
    L = Q + n + Wp + 2
    base = Q - Wp - 1

    def body(x_ref, *refs):
        wb = refs[: 2 * nconv]
        o_ref = refs[2 * nconv]
        scratch = refs[2 * nconv + 1:]

        def conv(load, w_ref, b_ref):
            taps = [load(base + dh * Wp + dw)
                    for dh in range(3) for dw in range(3)]
            cat = jnp.concatenate(taps, axis=1)
            y = jnp.dot(cat, w_ref[...], preferred_element_type=jnp.float32)
            y = jnp.maximum(y + b_ref[...], 0.0)
            colmask = (lax.broadcasted_iota(jnp.int32, y.shape, 0) % Wp) < W
            return jnp.where(colmask, y, 0.0).astype(_BF)

        load = lambda s: x_ref[0, pl.ds(s, n), :]
        for i in range(nconv):
            y = conv(load, wb[2 * i], wb[2 * i + 1])
            if i < nconv - 1:
                s_ref = scratch[i]
                c = y.shape[1]
                s_ref[0:Q, :] = jnp.zeros((Q, c), _BF)
                s_ref[pl.ds(Q, n), :] = y
                s_ref[pl.ds(Q + n, L - Q - n), :] = jnp.zeros(
                    (L - Q - n, c), _BF)
                load = (lambda r: lambda s: r[pl.ds(s, n), :])(s_ref)
            else:
                o_ref[0] = y

    return body, n, L


def _conv_stage(xf, wbs, *, H, W, Q):
    B, Lx, Cin = xf.shape
    nconv = len(wbs)
    body, n, L = _make_stage_body(H, W, Q, nconv)
    assert Lx == L, (Lx, L)
    couts = [w.shape[1] for w, _ in wbs]
    in_specs = [pl.BlockSpec((1, L, Cin), lambda b: (b, 0, 0))]
    args = [xf]
    for w, bias in wbs:
        in_specs.append(pl.BlockSpec(w.shape, lambda b: (0, 0)))
        in_specs.append(pl.BlockSpec(bias.shape, lambda b: (0, 0)))
        args += [w, bias]
    return pl.pallas_call(
        body,
        out_shape=jax.ShapeDtypeStruct((B, n, couts[-1]), _BF),
        grid=(B,),
        in_specs=in_specs,
        out_specs=pl.BlockSpec((1, n, couts[-1]), lambda b: (b, 0, 0)),
        scratch_shapes=[pltpu.VMEM((L, c), _BF) for c in couts[:-1]],
        compiler_params=pltpu.CompilerParams(
            dimension_semantics=("parallel",), vmem_limit_bytes=_VMEM),
    )(*args)


def _pool_body(x_ref, o_ref, *, C, Wo):
    r = jnp.maximum(x_ref[0, :, 0, :, :], x_ref[0, :, 1, :, :])
    m = jnp.maximum(r[:, :, :C], r[:, :, C:])
    o_ref[0] = m[:, :Wo, :]


def _pool(yf, *, H, W, C):
    B = yf.shape[0]
    Wp = W + 2
    Ho, Wo, Wpo = H // 2, W // 2, Wp // 2
    x5 = yf.reshape(B, Ho, 2, Wpo, 2 * C)
    return pl.pallas_call(
        functools.partial(_pool_body, C=C, Wo=Wo),
        out_shape=jax.ShapeDtypeStruct((B, Ho, Wo, C), _BF),
        grid=(B,),
        in_specs=[pl.BlockSpec((1, Ho, 2, Wpo, 2 * C),
                               lambda b: (b, 0, 0, 0, 0))],
        out_specs=pl.BlockSpec((1, Ho, Wo, C), lambda b: (b, 0, 0, 0)),
        compiler_params=pltpu.CompilerParams(
            dimension_semantics=("parallel",), vmem_limit_bytes=_VMEM),
    )(x5)


def _fc1_body(x_ref, w_ref, b_ref, o_ref, acc_ref):
    k = pl.program_id(1)

    @pl.when(k == 0)
    def _():
        acc_ref[...] = jnp.zeros_like(acc_ref)

    acc_ref[...] += jnp.dot(x_ref[...], w_ref[...],
                            preferred_element_type=jnp.float32)

    @pl.when(k == pl.num_programs(1) - 1)
    def _():
        o_ref[...] = jnp.maximum(acc_ref[...] + b_ref[...], 0.0)


def _fc1(x, w, b):
    B, K = x.shape
    N = w.shape[1]
    tk, tn = 2048, N // 2
    grid = (2, K // tk)
    return pl.pallas_call(
        _fc1_body,
        out_shape=jax.ShapeDtypeStruct((B, N), jnp.float32),
        grid=grid,
        in_specs=[
            pl.BlockSpec((B, tk), lambda j, k: (0, k)),
            pl.BlockSpec((tk, tn), lambda j, k: (k, j)),
            pl.BlockSpec((1, tn), lambda j, k: (0, j)),
        ],
        out_specs=pl.BlockSpec((B, tn), lambda j, k: (0, j)),
        scratch_shapes=[pltpu.VMEM((B, tn), jnp.float32)],
        compiler_params=pltpu.CompilerParams(
            dimension_semantics=("parallel", "arbitrary"),
            vmem_limit_bytes=_VMEM),
    )(x, w, b)


def _fc23_body(x_ref, w2_ref, b2_ref, w3_ref, o_ref, acc_ref):
    k = pl.program_id(1)

    @pl.when(k == 0)
    def _():
        acc_ref[...] = jnp.zeros_like(acc_ref)

    acc_ref[...] += jnp.dot(x_ref[...], w2_ref[...],
                            preferred_element_type=jnp.float32)

    @pl.when(k == pl.num_programs(1) - 1)
    def _():
        h = jnp.maximum(acc_ref[...] + b2_ref[...], 0.0)
        o_ref[0] = jnp.dot(h, w3_ref[...],
                           preferred_element_type=jnp.float32)


def _fc23(x, w2, b2, w3):
    B, K = x.shape
    N = w2.shape[1]
    NC = w3.shape[1]
    tk, tn = 1024, N // 2
    grid = (2, K // tk)
    return pl.pallas_call(
        _fc23_body,
        out_shape=jax.ShapeDtypeStruct((2, B, NC), jnp.float32),
        grid=grid,
        in_specs=[
            pl.BlockSpec((B, tk), lambda j, k: (0, k)),
            pl.BlockSpec((tk, tn), lambda j, k: (k, j)),
            pl.BlockSpec((1, tn), lambda j, k: (0, j)),
            pl.BlockSpec((tn, NC), lambda j, k: (j, 0)),
        ],
        out_specs=pl.BlockSpec((1, B, NC), lambda j, k: (j, 0, 0)),
        scratch_shapes=[pltpu.VMEM((B, tn), jnp.float32)],
        compiler_params=pltpu.CompilerParams(
            dimension_semantics=("parallel", "arbitrary"),
            vmem_limit_bytes=_VMEM),
    )(x, w2, b2, w3)


def _finalize_body(z_ref, b_ref, o_ref):
    z = z_ref[0] + z_ref[1] + b_ref[...]
    z = z - jnp.max(z, axis=-1, keepdims=True)
    e = jnp.exp(z)
    o_ref[...] = e / jnp.sum(e, axis=-1, keepdims=True)


def _finalize(zp, b3):
    _, B, NC = zp.shape
    return pl.pallas_call(
        _finalize_body,
        out_shape=jax.ShapeDtypeStruct((B, NC), jnp.float32),
        grid=(1,),
        in_specs=[pl.BlockSpec((2, B, NC), lambda i: (0, 0, 0)),
                  pl.BlockSpec((1, NC), lambda i: (0, 0))],
        out_specs=pl.BlockSpec((B, NC), lambda i: (0, 0)),
        compiler_params=pltpu.CompilerParams(
            dimension_semantics=("arbitrary",), vmem_limit_bytes=_VMEM),
    )(zp, b3)


_STAGES = [(96, 112, 2), (48, 64, 2), (24, 32, 3), (12, 16, 3)]


def kernel(x_nchw,
           conv10_w, conv10_b, conv11_w, conv11_b,
           conv20_w, conv20_b, conv21_w, conv21_b,
           conv30_w, conv30_b, conv31_w, conv31_b, conv32_w, conv32_b,
           conv40_w, conv40_b, conv41_w, conv41_b, conv42_w, conv42_b,
           fc1_w, fc1_b, fc2_w, fc2_b, fc3_w, fc3_b):
    B = x_nchw.shape[0]
    conv_wbs = [
        [(conv10_w, conv10_b), (conv11_w, conv11_b)],
        [(conv20_w, conv20_b), (conv21_w, conv21_b)],
        [(conv30_w, conv30_b), (conv31_w, conv31_b), (conv32_w, conv32_b)],
        [(conv40_w, conv40_b), (conv41_w, conv41_b), (conv42_w, conv42_b)],
    ]
    x = jnp.transpose(x_nchw, (0, 2, 3, 1)).astype(_BF)
    for (hw, Q, nconv), wbs in zip(_STAGES, conv_wbs):
        Wp = hw + 2
        L = Q + hw * Wp + Wp + 2
        xf = _flat(x, Wp, Q, L)
        wbs9 = [(w.astype(_BF).reshape(9 * w.shape[2], w.shape[3]), bias)
                for w, bias in wbs]
        y = _conv_stage(xf, wbs9, H=hw, W=hw, Q=Q)
        x = _pool(y, H=hw, W=hw, C=wbs9[-1][0].shape[1])
    feat = jnp.transpose(x, (0, 3, 1, 2)).reshape(B, -1).astype(jnp.float32)
    h1 = _fc1(feat, fc1_w, fc1_b)
    zp = _fc23(h1, fc2_w, fc2_b, fc3_w)
    return _finalize(zp, fc3_b)


# speedup vs baseline: 1.3215x; 1.1380x over previous
import functools

import jax
import jax.numpy as jnp
from jax import lax
from jax.experimental import pallas as pl
from jax.experimental.pallas import tpu as pltpu

_VMEM = 56 << 20
_BF = jnp.bfloat16

_STAGES = [
    dict(H=96, Wp=112, Win=96, nconv=2),
    dict(H=48, Wp=64, Win=56, nconv=2),
    dict(H=24, Wp=32, Win=32, nconv=3),
    dict(H=12, Wp=16, Win=16, nconv=3),
]


def _make_stage_body(H, Wp, Win, nconv, cins, couts):
    W = H
    n = H * Wp
    P = Wp + 1
    L = n + 2 * Wp
    Ho, Wpo = H // 2, Wp // 2

    def body(x_ref, *refs):
        wb = refs[: 2 * nconv]
        o_ref = refs[2 * nconv]
        scratch = refs[2 * nconv + 1: 2 * nconv + 1 + nconv]
        ps = refs[-1]

        x0 = x_ref[0]
        if Win < Wp:
            x0 = jnp.concatenate(
                [x0, jnp.zeros((H, Wp - Win, cins[0]), _BF)], axis=1)
        v = x0.reshape(n, cins[0])

        mask = (lax.broadcasted_iota(jnp.int32, (n, couts[0]), 0) % Wp) < W
        y = None
        for i in range(nconv):
            scr = scratch[i]
            c = cins[i]
            scr[0:P, :] = jnp.zeros((P, 3 * c), _BF)
            scr[pl.ds(P + n - 2, Wp + 1), :] = jnp.zeros((Wp + 1, 3 * c), _BF)
            for dw in range(3):
                scr[pl.ds(P - dw, n), dw * c:(dw + 1) * c] = v
            w_ref, b_ref = wb[2 * i], wb[2 * i + 1]
            y = jnp.dot(scr[pl.ds(0, n), :], w_ref[0],
                        preferred_element_type=jnp.float32)
            y += jnp.dot(scr[pl.ds(Wp, n), :], w_ref[1],
                         preferred_element_type=jnp.float32)
            y += jnp.dot(scr[pl.ds(2 * Wp, n), :], w_ref[2],
                         preferred_element_type=jnp.float32)
            y = jnp.maximum(y + b_ref[...], 0.0)
            y = jnp.where(mask, y, 0.0)
            if i < nconv - 1:
                v = y.astype(_BF)

        cl = couts[-1]
        y3 = y.reshape(Ho, 2 * Wp, cl)
        r2 = jnp.maximum(y3[:, :Wp, :], y3[:, Wp:, :]).reshape(n // 2, cl)
        G = ps.shape[1]
        parts = []
        for g in range(G):
            gw = min(128, cl - g * 128)
            ps[:, g, 0:gw] = r2[:, g * 128:g * 128 + gw]
            parts.append(jnp.maximum(ps[pl.Slice(0, n // 4, 2), g, 0:gw],
                                     ps[pl.Slice(1, n // 4, 2), g, 0:gw]))
        m = jnp.concatenate(parts, axis=-1) if G > 1 else parts[0]
        o_ref[0] = m.reshape(Ho, Wpo, cl).astype(_BF)

    return body, n, L, Ho, Wpo


def _conv_stage(x, wbs, *, H, Wp, Win, nconv):
    B = x.shape[0]
    cins = [w.shape[1] // 3 for w, _ in wbs]
    couts = [w.shape[2] for w, _ in wbs]
    body, n, L, Ho, Wpo = _make_stage_body(H, Wp, Win, nconv, cins, couts)
    in_specs = [pl.BlockSpec((1, H, Win, cins[0]), lambda b: (b, 0, 0, 0))]
    args = [x]
    for w, bias in wbs:
        in_specs.append(pl.BlockSpec(w.shape, lambda b: (0, 0, 0)))
        in_specs.append(pl.BlockSpec(bias.shape, lambda b: (0, 0)))
        args += [w, bias]
    scratch = [pltpu.VMEM((L, 3 * c), _BF) for c in cins]
    scratch.append(
        pltpu.VMEM((n // 2, (couts[-1] + 127) // 128, 128), jnp.float32))
    return pl.pallas_call(
        body,
        out_shape=jax.ShapeDtypeStruct((B, Ho, Wpo, couts[-1]), _BF),
        grid=(B,),
        in_specs=in_specs,
        out_specs=pl.BlockSpec((1, Ho, Wpo, couts[-1]),
                               lambda b: (b, 0, 0, 0)),
        scratch_shapes=scratch,
        compiler_params=pltpu.CompilerParams(
            dimension_semantics=("parallel",), vmem_limit_bytes=_VMEM),
    )(*args)


def _fc1_body(x_ref, w_ref, b_ref, o_ref, acc_ref):
    k = pl.program_id(1)

    @pl.when(k == 0)
    def _():
        acc_ref[...] = jnp.zeros_like(acc_ref)

    acc_ref[...] += jnp.dot(x_ref[...], w_ref[...],
                            preferred_element_type=jnp.float32)

    @pl.when(k == pl.num_programs(1) - 1)
    def _():
        o_ref[...] = jnp.maximum(acc_ref[...] + b_ref[...], 0.0)


def _fc1(x, w, b):
    B, K = x.shape
    N = w.shape[1]
    tk, tn = 2048, N // 2
    return pl.pallas_call(
        _fc1_body,
        out_shape=jax.ShapeDtypeStruct((B, N), jnp.float32),
        grid=(2, K // tk),
        in_specs=[
            pl.BlockSpec((B, tk), lambda j, k: (0, k)),
            pl.BlockSpec((tk, tn), lambda j, k: (k, j)),
            pl.BlockSpec((1, tn), lambda j, k: (0, j)),
        ],
        out_specs=pl.BlockSpec((B, tn), lambda j, k: (0, j)),
        scratch_shapes=[pltpu.VMEM((B, tn), jnp.float32)],
        compiler_params=pltpu.CompilerParams(
            dimension_semantics=("parallel", "arbitrary"),
            vmem_limit_bytes=_VMEM),
    )(x, w, b)


def _fc23_body(x_ref, w2_ref, b2_ref, w3_ref, o_ref, acc_ref):
    k = pl.program_id(1)

    @pl.when(k == 0)
    def _():
        acc_ref[...] = jnp.zeros_like(acc_ref)

    acc_ref[...] += jnp.dot(x_ref[...], w2_ref[...],
                            preferred_element_type=jnp.float32)

    @pl.when(k == pl.num_programs(1) - 1)
    def _():
        h = jnp.maximum(acc_ref[...] + b2_ref[...], 0.0)
        o_ref[0] = jnp.dot(h, w3_ref[...],
                           preferred_element_type=jnp.float32)


def _fc23(x, w2, b2, w3):
    B, K = x.shape
    N = w2.shape[1]
    NC = w3.shape[1]
    tk, tn = 1024, N // 2
    return pl.pallas_call(
        _fc23_body,
        out_shape=jax.ShapeDtypeStruct((2, B, NC), jnp.float32),
        grid=(2, K // tk),
        in_specs=[
            pl.BlockSpec((B, tk), lambda j, k: (0, k)),
            pl.BlockSpec((tk, tn), lambda j, k: (k, j)),
            pl.BlockSpec((1, tn), lambda j, k: (0, j)),
            pl.BlockSpec((tn, NC), lambda j, k: (j, 0)),
        ],
        out_specs=pl.BlockSpec((1, B, NC), lambda j, k: (j, 0, 0)),
        scratch_shapes=[pltpu.VMEM((B, tn), jnp.float32)],
        compiler_params=pltpu.CompilerParams(
            dimension_semantics=("parallel", "arbitrary"),
            vmem_limit_bytes=_VMEM),
    )(x, w2, b2, w3)


def _finalize_body(z_ref, b_ref, o_ref):
    z = z_ref[0] + z_ref[1] + b_ref[...]
    z = z - jnp.max(z, axis=-1, keepdims=True)
    e = jnp.exp(z)
    o_ref[...] = e / jnp.sum(e, axis=-1, keepdims=True)


def _finalize(zp, b3):
    _, B, NC = zp.shape
    return pl.pallas_call(
        _finalize_body,
        out_shape=jax.ShapeDtypeStruct((B, NC), jnp.float32),
        grid=(1,),
        in_specs=[pl.BlockSpec((2, B, NC), lambda i: (0, 0, 0)),
                  pl.BlockSpec((1, NC), lambda i: (0, 0))],
        out_specs=pl.BlockSpec((B, NC), lambda i: (0, 0)),
        compiler_params=pltpu.CompilerParams(
            dimension_semantics=("arbitrary",), vmem_limit_bytes=_VMEM),
    )(zp, b3)


def kernel(x_nchw,
           conv10_w, conv10_b, conv11_w, conv11_b,
           conv20_w, conv20_b, conv21_w, conv21_b,
           conv30_w, conv30_b, conv31_w, conv31_b, conv32_w, conv32_b,
           conv40_w, conv40_b, conv41_w, conv41_b, conv42_w, conv42_b,
           fc1_w, fc1_b, fc2_w, fc2_b, fc3_w, fc3_b):
    B = x_nchw.shape[0]
    conv_wbs = [
        [(conv10_w, conv10_b), (conv11_w, conv11_b)],
        [(conv20_w, conv20_b), (conv21_w, conv21_b)],
        [(conv30_w, conv30_b), (conv31_w, conv31_b), (conv32_w, conv32_b)],
        [(conv40_w, conv40_b), (conv41_w, conv41_b), (conv42_w, conv42_b)],
    ]
    x = jnp.transpose(x_nchw, (0, 2, 3, 1)).astype(_BF)
    for cfg, wbs in zip(_STAGES, conv_wbs):
        wbs3 = [(w.astype(_BF).reshape(3, 3 * w.shape[2], w.shape[3]), bias)
                for w, bias in wbs]
        x = _conv_stage(x, wbs3, **cfg)
    feat = jnp.transpose(x[:, :, :6, :], (0, 3, 1, 2)).reshape(B, -1)
    feat = feat.astype(jnp.float32)
    h1 = _fc1(feat, fc1_w, fc1_b)
    zp = _fc23(h1, fc2_w, fc2_b, fc3_w)
    return _finalize(zp, fc3_b)
